# baseline (speedup 1.0000x reference)
import functools

import jax
import jax.numpy as jnp
from jax import lax
from jax.experimental import pallas as pl
from jax.experimental.pallas import tpu as pltpu
from jax.experimental.pallas import tpu_sc as plsc

_NC = 2
_NS = 16
_NW = _NC * _NS
_U = 128


def _sc_scores(ctab, xtab, cidx, xidx, nidx, B, BN, D, neg):
    uc = B // _U // _NW
    un = BN // _U // _NW
    nd16 = D // 16
    rc = uc * _U
    rn = un * _U

    mesh = plsc.VectorSubcoreMesh(core_axis_name="c", subcore_axis_name="s")

    @functools.partial(
        pl.kernel,
        mesh=mesh,
        out_type=[
            jax.ShapeDtypeStruct((B,), jnp.float32),
            jax.ShapeDtypeStruct((BN,), jnp.float32),
        ],
        scratch_types=[
            pltpu.VMEM((_U,), jnp.int32),
            pltpu.VMEM((_U,), jnp.int32),
            pltpu.VMEM((rc, D), jnp.float32),
            pltpu.VMEM((_U, D), jnp.float32),
            pltpu.VMEM((_U, D), jnp.float32),
            pltpu.VMEM((rc,), jnp.float32),
            pltpu.VMEM((rn,), jnp.float32),
            pltpu.SemaphoreType.DMA,
        ],
        compiler_params=pltpu.CompilerParams(use_tc_tiling_on_sc=False,
                                             needs_layout_passes=False),
    )
    def sc_fn(ctab_h, xtab_h, cidx_h, xidx_h, nidx_h,
              pos_h, negout_h,
              idx0, idx1, cbuf, bufa, bufb, pbuf, nbuf, sem):
        wid = lax.axis_index("s") * _NC + lax.axis_index("c")
        iota = lax.iota(jnp.int32, 16)
        zero16 = jnp.zeros((16,), jnp.float32)

        def gather_unit(tab_h, idx_h, u):
            pltpu.sync_copy(idx_h.at[0, u], idx0)
            pltpu.sync_copy(idx_h.at[1, u], idx1)
            cp0 = pltpu.async_copy(tab_h.at[idx0], bufa, sem)
            cp1 = pltpu.async_copy(tab_h.at[idx1], bufb, sem)
            cp0.wait()
            cp1.wait()

        def cunit(j, carry):
            gather_unit(ctab_h, cidx_h, wid * uc + j)

            def srow(r, c2):
                for c in range(nd16):
                    sl = pl.ds(c * 16, 16)
                    cbuf[j * _U + r, sl] = bufa[r, sl] + bufb[r, sl]
                return c2

            lax.fori_loop(0, _U, srow, 0)
            return carry

        lax.fori_loop(0, uc, cunit, 0)

        def xunit(j, carry):
            gather_unit(xtab_h, xidx_h, wid * uc + j)

            def srow(r, c2):
                for c in range(nd16):
                    sl = pl.ds(c * 16, 16)
                    bufa[r, sl] = bufa[r, sl] + bufb[r, sl]
                return c2

            lax.fori_loop(0, _U, srow, 0)

            def dotg(g, c2):
                rows_u = g * 16 + iota
                rows_c = j * _U + rows_u

                def dd(d4, acc):
                    for k in range(4):
                        dv = jnp.full((16,), 0, jnp.int32) + (d4 * 4 + k)
                        va = plsc.load_gather(cbuf, [rows_c, dv])
                        vb = plsc.load_gather(bufa, [rows_u, dv])
                        acc = acc + va * vb
                    return acc

                acc = lax.fori_loop(0, D // 4, dd, zero16)
                pbuf[pl.ds(j * _U + g * 16, 16)] = acc
                return c2

            lax.fori_loop(0, _U // 16, dotg, 0)
            return carry

        lax.fori_loop(0, uc, xunit, 0)
        pltpu.sync_copy(pbuf, pos_h.at[pl.ds(wid * rc, rc)])

        def nunit(j, carry):
            gather_unit(xtab_h, nidx_h, wid * un + j)

            def srow(r, c2):
                for c in range(nd16):
                    sl = pl.ds(c * 16, 16)
                    bufa[r, sl] = bufa[r, sl] + bufb[r, sl]
                return c2

            lax.fori_loop(0, _U, srow, 0)

            def dotg(g, c2):
                rows_u = g * 16 + iota
                rows_c = (j * _U + rows_u) // neg

                def dd(d4, acc):
                    for k in range(4):
                        dv = jnp.full((16,), 0, jnp.int32) + (d4 * 4 + k)
                        va = plsc.load_gather(cbuf, [rows_c, dv])
                        vb = plsc.load_gather(bufa, [rows_u, dv])
                        acc = acc + va * vb
                    return acc

                acc = lax.fori_loop(0, D // 4, dd, zero16)
                nbuf[pl.ds(j * _U + g * 16, 16)] = acc
                return c2

            lax.fori_loop(0, _U // 16, dotg, 0)
            return carry

        lax.fori_loop(0, un, nunit, 0)
        pltpu.sync_copy(nbuf, negout_h.at[pl.ds(wid * rn, rn)])

    return sc_fn(ctab, xtab, cidx, xidx, nidx)


def _tc_loss(pos2d, neg2d, B):
    s = 0.0625

    def tc_fn(p_ref, n_ref, o_ref):
        ps = jnp.clip(s * p_ref[...], -10.0, 10.0)
        ns = jnp.clip(s * n_ref[...], -10.0, 10.0)
        pos_loss = jnp.sum(jnp.log(1.0 + jnp.exp(-ps)))
        neg_loss = jnp.sum(jnp.log(1.0 + jnp.exp(ns)))
        o_ref[0, 0] = (pos_loss + neg_loss) * (1.0 / B)

    out = pl.pallas_call(
        tc_fn,
        out_specs=pl.BlockSpec(memory_space=pltpu.SMEM),
        out_shape=jax.ShapeDtypeStruct((1, 1), jnp.float32),
    )(pos2d, neg2d)
    return out[0, 0]


def kernel(centers, contexts, neg_contexts, center_emb, context_emb):
    F, V, D = center_emb.shape
    B = centers.shape[0]
    BN = neg_contexts.shape[0]
    neg = BN // B

    ctab = center_emb.reshape(F * V, D)
    xtab = context_emb.reshape(F * V, D)
    offs = jnp.arange(F, dtype=jnp.int32) * V
    cidx = (centers + offs[None, :]).T.reshape(F, B // _U, _U)
    xidx = (contexts + offs[None, :]).T.reshape(F, B // _U, _U)
    nidx = (neg_contexts + offs[None, :]).T.reshape(F, BN // _U, _U)

    pos, negraw = _sc_scores(ctab, xtab, cidx, xidx, nidx, B, BN, D, neg)
    return _tc_loss(pos.reshape(B // 128, 128), negraw.reshape(BN // 128, 128), B)


# speedup vs baseline: 1.2571x; 1.2571x over previous
import functools

import jax
import jax.numpy as jnp
from jax import lax
from jax.experimental import pallas as pl
from jax.experimental.pallas import tpu as pltpu
from jax.experimental.pallas import tpu_sc as plsc

_NC = 2
_NS = 16
_NW = _NC * _NS
_U = 128


def _sc_scores(ctab, xtab, cidx, xidx, nidx, B, BN, D, neg):
    uc = B // _U // _NW
    un = BN // _U // _NW
    nd16 = D // 16
    rc = uc * _U
    rn = un * _U

    mesh = plsc.VectorSubcoreMesh(core_axis_name="c", subcore_axis_name="s")

    @functools.partial(
        pl.kernel,
        mesh=mesh,
        out_type=[
            jax.ShapeDtypeStruct((B,), jnp.float32),
            jax.ShapeDtypeStruct((BN,), jnp.float32),
        ],
        scratch_types=[
            pltpu.VMEM((_U,), jnp.int32),
            pltpu.VMEM((_U,), jnp.int32),
            pltpu.VMEM((rc, D + 1), jnp.float32),
            pltpu.VMEM((_U, D), jnp.float32),
            pltpu.VMEM((_U, D), jnp.float32),
            pltpu.VMEM((_U, D + 1), jnp.float32),
            pltpu.VMEM((rc,), jnp.float32),
            pltpu.VMEM((rn,), jnp.float32),
            pltpu.SemaphoreType.DMA,
        ],
        compiler_params=pltpu.CompilerParams(use_tc_tiling_on_sc=False,
                                             needs_layout_passes=False),
    )
    def sc_fn(ctab_h, xtab_h, cidx_h, xidx_h, nidx_h,
              pos_h, negout_h,
              idx0, idx1, cbuf, bufa, bufb, xpad, pbuf, nbuf, sem):
        wid = lax.axis_index("s") * _NC + lax.axis_index("c")
        iota = lax.iota(jnp.int32, 16)
        zero16 = jnp.zeros((16,), jnp.float32)

        def gather_unit(tab_h, idx_h, u):
            pltpu.sync_copy(idx_h.at[0, u], idx0)
            pltpu.sync_copy(idx_h.at[1, u], idx1)
            cp0 = pltpu.async_copy(tab_h.at[idx0], bufa, sem)
            cp1 = pltpu.async_copy(tab_h.at[idx1], bufb, sem)
            cp0.wait()
            cp1.wait()

        def cunit(j, carry):
            gather_unit(ctab_h, cidx_h, wid * uc + j)

            def srow(r, c2):
                for c in range(nd16):
                    sl = pl.ds(c * 16, 16)
                    cbuf[j * _U + r, sl] = bufa[r, sl] + bufb[r, sl]
                return c2

            lax.fori_loop(0, _U, srow, 0)
            return carry

        lax.fori_loop(0, uc, cunit, 0)

        def xunit(j, carry):
            gather_unit(xtab_h, xidx_h, wid * uc + j)

            def srow(r, c2):
                for c in range(nd16):
                    sl = pl.ds(c * 16, 16)
                    xpad[r, sl] = bufa[r, sl] + bufb[r, sl]
                return c2

            lax.fori_loop(0, _U, srow, 0)

            def dotg(g, c2):
                rows_u = g * 16 + iota
                rows_c = j * _U + rows_u

                def dd(d4, acc):
                    for k in range(4):
                        dv = jnp.full((16,), 0, jnp.int32) + (d4 * 4 + k)
                        va = plsc.load_gather(cbuf, [rows_c, dv])
                        vb = plsc.load_gather(xpad, [rows_u, dv])
                        acc = acc + va * vb
                    return acc

                acc = lax.fori_loop(0, D // 4, dd, zero16)
                pbuf[pl.ds(j * _U + g * 16, 16)] = acc
                return c2

            lax.fori_loop(0, _U // 16, dotg, 0)
            return carry

        lax.fori_loop(0, uc, xunit, 0)
        pltpu.sync_copy(pbuf, pos_h.at[pl.ds(wid * rc, rc)])

        def nunit(j, carry):
            gather_unit(xtab_h, nidx_h, wid * un + j)

            def srow(r, c2):
                for c in range(nd16):
                    sl = pl.ds(c * 16, 16)
                    xpad[r, sl] = bufa[r, sl] + bufb[r, sl]
                return c2

            lax.fori_loop(0, _U, srow, 0)

            def dotg(g, c2):
                rows_u = g * 16 + iota
                rows_c = (j * _U + rows_u) // neg

                def dd(d4, acc):
                    for k in range(4):
                        dv = jnp.full((16,), 0, jnp.int32) + (d4 * 4 + k)
                        va = plsc.load_gather(cbuf, [rows_c, dv])
                        vb = plsc.load_gather(xpad, [rows_u, dv])
                        acc = acc + va * vb
                    return acc

                acc = lax.fori_loop(0, D // 4, dd, zero16)
                nbuf[pl.ds(j * _U + g * 16, 16)] = acc
                return c2

            lax.fori_loop(0, _U // 16, dotg, 0)
            return carry

        lax.fori_loop(0, un, nunit, 0)
        pltpu.sync_copy(nbuf, negout_h.at[pl.ds(wid * rn, rn)])

    return sc_fn(ctab, xtab, cidx, xidx, nidx)


def _tc_loss(pos2d, neg2d, B):
    s = 0.0625

    def tc_fn(p_ref, n_ref, o_ref):
        ps = jnp.clip(s * p_ref[...], -10.0, 10.0)
        ns = jnp.clip(s * n_ref[...], -10.0, 10.0)
        pos_loss = jnp.sum(jnp.log(1.0 + jnp.exp(-ps)))
        neg_loss = jnp.sum(jnp.log(1.0 + jnp.exp(ns)))
        o_ref[0, 0] = (pos_loss + neg_loss) * (1.0 / B)

    out = pl.pallas_call(
        tc_fn,
        out_specs=pl.BlockSpec(memory_space=pltpu.SMEM),
        out_shape=jax.ShapeDtypeStruct((1, 1), jnp.float32),
    )(pos2d, neg2d)
    return out[0, 0]


def kernel(centers, contexts, neg_contexts, center_emb, context_emb):
    F, V, D = center_emb.shape
    B = centers.shape[0]
    BN = neg_contexts.shape[0]
    neg = BN // B

    ctab = center_emb.reshape(F * V, D)
    xtab = context_emb.reshape(F * V, D)
    offs = jnp.arange(F, dtype=jnp.int32) * V
    cidx = (centers + offs[None, :]).T.reshape(F, B // _U, _U)
    xidx = (contexts + offs[None, :]).T.reshape(F, B // _U, _U)
    nidx = (neg_contexts + offs[None, :]).T.reshape(F, BN // _U, _U)

    pos, negraw = _sc_scores(ctab, xtab, cidx, xidx, nidx, B, BN, D, neg)
    return _tc_loss(pos.reshape(B // 128, 128), negraw.reshape(BN // 128, 128), B)


# speedup vs baseline: 2.2580x; 1.7962x over previous
import functools

import jax
import jax.numpy as jnp
from jax import lax
from jax.experimental import pallas as pl
from jax.experimental.pallas import tpu as pltpu
from jax.experimental.pallas import tpu_sc as plsc

_NC = 2
_NS = 16
_NW = _NC * _NS
_U = 128


def _sc_scores(ctab, xtab, cidx, xidx, nidx, B, BN, D, neg):
    uc = B // _U // _NW
    un = BN // _U // _NW
    nd16 = D // 16
    rc = uc * _U
    rn = un * _U
    w = 2 * D

    mesh = plsc.VectorSubcoreMesh(core_axis_name="c", subcore_axis_name="s")

    @functools.partial(
        pl.kernel,
        mesh=mesh,
        out_type=[
            jax.ShapeDtypeStruct((B,), jnp.float32),
            jax.ShapeDtypeStruct((BN,), jnp.float32),
        ],
        scratch_types=[
            pltpu.VMEM((_U,), jnp.int32),
            pltpu.VMEM((_U,), jnp.int32),
            pltpu.VMEM((rc, D), jnp.float32),
            pltpu.VMEM((_U, w), jnp.float32),
            pltpu.VMEM((_U, w), jnp.float32),
            pltpu.VMEM((rc,), jnp.float32),
            pltpu.VMEM((rn,), jnp.float32),
            pltpu.SemaphoreType.DMA,
        ],
        compiler_params=pltpu.CompilerParams(use_tc_tiling_on_sc=True,
                                             needs_layout_passes=False),
    )
    def sc_fn(ctab_h, xtab_h, cidx_h, xidx_h, nidx_h,
              pos_h, negout_h,
              idx0, idx1, cbuf, bufa, bufb, pbuf, nbuf, sem):
        wid = lax.axis_index("s") * _NC + lax.axis_index("c")
        iota = lax.iota(jnp.int32, 16)
        zero16 = jnp.zeros((16,), jnp.float32)

        def gather_unit(tab_h, idx_h, u):
            pltpu.sync_copy(idx_h.at[0, u], idx0)
            pltpu.sync_copy(idx_h.at[1, u], idx1)
            cp0 = pltpu.async_copy(tab_h.at[idx0], bufa, sem)
            cp1 = pltpu.async_copy(tab_h.at[idx1], bufb, sem)
            cp0.wait()
            cp1.wait()


        def cunit(j, carry):
            gather_unit(ctab_h, cidx_h, wid * uc + j)

            def srow(r, c2):
                for c in range(nd16):
                    sl = pl.ds(c * 16, 16)
                    sl1 = pl.ds(D + c * 16, 16)
                    cbuf[j * _U + r, sl] = bufa[r, sl] + bufb[r, sl1]
                return c2

            lax.fori_loop(0, _U, srow, 0)
            return carry

        lax.fori_loop(0, uc, cunit, 0)

        def dot_groups(j, rows_c_of, out_buf):
            def dotg(g, c2):
                rows_u = g * 16 + iota
                rows_c = rows_c_of(j * _U + rows_u)

                def dd(k4, acc):
                    for kk in range(4):
                        dcol = jnp.bitwise_and(iota + (k4 * 4 + kk), D - 1)
                        va = plsc.load_gather(cbuf, [rows_c, dcol])
                        xa = plsc.load_gather(bufa, [rows_u, dcol])
                        xb = plsc.load_gather(bufb, [rows_u, dcol + D])
                        acc = acc + va * (xa + xb)
                    return acc

                acc = lax.fori_loop(0, D // 4, dd, zero16)
                out_buf[pl.ds(j * _U + g * 16, 16)] = acc
                return c2

            lax.fori_loop(0, _U // 16, dotg, 0)

        def xunit(j, carry):
            gather_unit(xtab_h, xidx_h, wid * uc + j)
            dot_groups(j, lambda r: r, pbuf)
            return carry

        lax.fori_loop(0, uc, xunit, 0)
        pltpu.sync_copy(pbuf, pos_h.at[pl.ds(wid * rc, rc)])

        def nunit(j, carry):
            gather_unit(xtab_h, nidx_h, wid * un + j)
            dot_groups(j, lambda r: r // neg, nbuf)
            return carry

        lax.fori_loop(0, un, nunit, 0)
        pltpu.sync_copy(nbuf, negout_h.at[pl.ds(wid * rn, rn)])

    return sc_fn(ctab, xtab, cidx, xidx, nidx)


def _tc_loss(pos2d, neg2d, B):
    s = 0.0625

    def tc_fn(p_ref, n_ref, o_ref):
        ps = jnp.clip(s * p_ref[...], -10.0, 10.0)
        ns = jnp.clip(s * n_ref[...], -10.0, 10.0)
        pos_loss = jnp.sum(jnp.log(1.0 + jnp.exp(-ps)))
        neg_loss = jnp.sum(jnp.log(1.0 + jnp.exp(ns)))
        o_ref[0, 0] = (pos_loss + neg_loss) * (1.0 / B)

    out = pl.pallas_call(
        tc_fn,
        out_specs=pl.BlockSpec(memory_space=pltpu.SMEM),
        out_shape=jax.ShapeDtypeStruct((1, 1), jnp.float32),
    )(pos2d, neg2d)
    return out[0, 0]


def kernel(centers, contexts, neg_contexts, center_emb, context_emb):
    F, V, D = center_emb.shape
    B = centers.shape[0]
    BN = neg_contexts.shape[0]
    neg = BN // B

    ctab = jnp.transpose(center_emb, (1, 0, 2)).reshape(V, F * D)
    xtab = jnp.transpose(context_emb, (1, 0, 2)).reshape(V, F * D)
    cidx = centers.T.reshape(F, B // _U, _U)
    xidx = contexts.T.reshape(F, B // _U, _U)
    nidx = neg_contexts.T.reshape(F, BN // _U, _U)

    pos, negraw = _sc_scores(ctab, xtab, cidx, xidx, nidx, B, BN, D, neg)
    return _tc_loss(pos.reshape(B // 128, 128), negraw.reshape(BN // 128, 128), B)


# speedup vs baseline: 2.8111x; 1.2450x over previous
import functools

import jax
import jax.numpy as jnp
from jax import lax
from jax.experimental import pallas as pl
from jax.experimental.pallas import tpu as pltpu
from jax.experimental.pallas import tpu_sc as plsc

_NC = 2
_NS = 16
_NW = _NC * _NS
_U = 64


def _sc_scores(ctab, xtab, cidx, xidx, nidx, B, BN, D, neg):
    uc = B // _U // _NW
    un = BN // _U // _NW
    nd16 = D // 16
    rc = uc * _U
    rn = un * _U
    w = 2 * D

    mesh = plsc.VectorSubcoreMesh(core_axis_name="c", subcore_axis_name="s")

    @functools.partial(
        pl.kernel,
        mesh=mesh,
        out_type=[
            jax.ShapeDtypeStruct((B,), jnp.float32),
            jax.ShapeDtypeStruct((BN,), jnp.float32),
        ],
        scratch_types=[
            pltpu.VMEM((2, uc, _U), jnp.int32),
            pltpu.VMEM((2, uc, _U), jnp.int32),
            pltpu.VMEM((2, un, _U), jnp.int32),
            pltpu.VMEM((rc, D), jnp.float32),
            pltpu.VMEM((_U, w), jnp.float32),
            pltpu.VMEM((_U, w), jnp.float32),
            pltpu.VMEM((_U, w), jnp.float32),
            pltpu.VMEM((_U, w), jnp.float32),
            pltpu.VMEM((rc,), jnp.float32),
            pltpu.VMEM((rn,), jnp.float32),
            pltpu.SemaphoreType.DMA,
            pltpu.SemaphoreType.DMA,
        ],
        compiler_params=pltpu.CompilerParams(use_tc_tiling_on_sc=True,
                                             needs_layout_passes=False),
    )
    def sc_fn(ctab_h, xtab_h, cidx_h, xidx_h, nidx_h,
              pos_h, negout_h,
              idxc, idxx, idxn, cbuf, ba0, bb0, ba1, bb1,
              pbuf, nbuf, sem0, sem1):
        wid = lax.axis_index("s") * _NC + lax.axis_index("c")
        iota = lax.iota(jnp.int32, 16)
        zero16 = jnp.zeros((16,), jnp.float32)
        bufs = ((ba0, bb0, sem0), (ba1, bb1, sem1))

        basec = pl.multiple_of(wid * uc, 8)
        basen = pl.multiple_of(wid * un, 8)
        for f in range(2):
            pltpu.sync_copy(cidx_h.at[f, pl.ds(basec, uc)], idxc.at[f])
            pltpu.sync_copy(xidx_h.at[f, pl.ds(basec, uc)], idxx.at[f])
            pltpu.sync_copy(nidx_h.at[f, pl.ds(basen, un)], idxn.at[f])

        def fire(tab_h, idxv, j, p):
            ba, bb, sem = bufs[p]
            pltpu.async_copy(tab_h.at[idxv.at[0, j]], ba, sem)
            pltpu.async_copy(tab_h.at[idxv.at[1, j]], bb, sem)

        def wait(tab_h, idxv, p):
            ba, bb, sem = bufs[p]
            pltpu.make_async_copy(tab_h.at[idxv.at[0, 0]], ba, sem).wait()
            pltpu.make_async_copy(tab_h.at[idxv.at[1, 0]], bb, sem).wait()

        def pipeline(tab_h, idxv, n_units, compute):
            fire(tab_h, idxv, 0, 0)

            def body(j2, carry):
                u = 2 * j2
                fire(tab_h, idxv, u + 1, 1)
                wait(tab_h, idxv, 0)
                compute(u, 0)
                fire(tab_h, idxv, jnp.minimum(u + 2, n_units - 1), 0)
                wait(tab_h, idxv, 1)
                compute(u + 1, 1)
                return carry

            lax.fori_loop(0, n_units // 2, body, 0)
            wait(tab_h, idxv, 0)

        def csum(j, p):
            ba, bb, _ = bufs[p]

            def srow(r, c2):
                for c in range(nd16):
                    sl = pl.ds(c * 16, 16)
                    sl1 = pl.ds(D + c * 16, 16)
                    cbuf[j * _U + r, sl] = ba[r, sl] + bb[r, sl1]
                return c2

            lax.fori_loop(0, _U, srow, 0)

        pipeline(ctab_h, idxc, uc, csum)

        def dots(j, p, rows_c_of, out_buf):
            ba, bb, _ = bufs[p]

            def dotg(g, c2):
                rows_u = g * 16 + iota
                rows_c = rows_c_of(j * _U + rows_u)

                def dd(k4, acc):
                    for kk in range(4):
                        dcol = jnp.bitwise_and(iota + (k4 * 4 + kk), D - 1)
                        va = plsc.load_gather(cbuf, [rows_c, dcol])
                        xa = plsc.load_gather(ba, [rows_u, dcol])
                        xb = plsc.load_gather(bb, [rows_u, dcol + D])
                        acc = acc + va * (xa + xb)
                    return acc

                acc = lax.fori_loop(0, D // 4, dd, zero16)
                out_buf[pl.ds(j * _U + g * 16, 16)] = acc
                return c2

            lax.fori_loop(0, _U // 16, dotg, 0)

        pipeline(xtab_h, idxx, uc,
                 lambda j, p: dots(j, p, lambda r: r, pbuf))
        pltpu.sync_copy(pbuf, pos_h.at[pl.ds(wid * rc, rc)])

        pipeline(xtab_h, idxn, un,
                 lambda j, p: dots(j, p, lambda r: r // neg, nbuf))
        pltpu.sync_copy(nbuf, negout_h.at[pl.ds(wid * rn, rn)])

    return sc_fn(ctab, xtab, cidx, xidx, nidx)


def _tc_loss(pos2d, neg2d, B):
    s = 0.0625

    def tc_fn(p_ref, n_ref, o_ref):
        ps = jnp.clip(s * p_ref[...], -10.0, 10.0)
        ns = jnp.clip(s * n_ref[...], -10.0, 10.0)
        pos_loss = jnp.sum(jnp.log(1.0 + jnp.exp(-ps)))
        neg_loss = jnp.sum(jnp.log(1.0 + jnp.exp(ns)))
        o_ref[0, 0] = (pos_loss + neg_loss) * (1.0 / B)

    out = pl.pallas_call(
        tc_fn,
        out_specs=pl.BlockSpec(memory_space=pltpu.SMEM),
        out_shape=jax.ShapeDtypeStruct((1, 1), jnp.float32),
    )(pos2d, neg2d)
    return out[0, 0]


def kernel(centers, contexts, neg_contexts, center_emb, context_emb):
    F, V, D = center_emb.shape
    B = centers.shape[0]
    BN = neg_contexts.shape[0]
    neg = BN // B

    ctab = jnp.transpose(center_emb, (1, 0, 2)).reshape(V, F * D)
    xtab = jnp.transpose(context_emb, (1, 0, 2)).reshape(V, F * D)
    cidx = centers.T.reshape(F, B // _U, _U)
    xidx = contexts.T.reshape(F, B // _U, _U)
    nidx = neg_contexts.T.reshape(F, BN // _U, _U)

    pos, negraw = _sc_scores(ctab, xtab, cidx, xidx, nidx, B, BN, D, neg)
    return _tc_loss(pos.reshape(B // 128, 128), negraw.reshape(BN // 128, 128), B)


# speedup vs baseline: 2.9492x; 1.0491x over previous
import functools

import jax
import jax.numpy as jnp
from jax import lax
from jax.experimental import pallas as pl
from jax.experimental.pallas import tpu as pltpu
from jax.experimental.pallas import tpu_sc as plsc

_NC = 2
_NS = 16
_NW = _NC * _NS
_U = 64

_MESH = dict(core_axis_name="c", subcore_axis_name="s")
_PARAMS = pltpu.CompilerParams(use_tc_tiling_on_sc=True,
                               needs_layout_passes=False)


def _mk_pipeline(bufs):
    def fire(tab_h, idxv, j, p):
        ba, bb, sem = bufs[p]
        pltpu.async_copy(tab_h.at[idxv.at[0, j]], ba, sem)
        pltpu.async_copy(tab_h.at[idxv.at[1, j]], bb, sem)

    def wait(tab_h, idxv, p):
        ba, bb, sem = bufs[p]
        pltpu.make_async_copy(tab_h.at[idxv.at[0, 0]], ba, sem).wait()
        pltpu.make_async_copy(tab_h.at[idxv.at[1, 0]], bb, sem).wait()

    def pipeline(tab_h, idxv, n_units, compute):
        fire(tab_h, idxv, 0, 0)

        def body(j2, carry):
            u = 2 * j2
            fire(tab_h, idxv, u + 1, 1)
            wait(tab_h, idxv, 0)
            compute(u, 0)
            fire(tab_h, idxv, jnp.minimum(u + 2, n_units - 1), 0)
            wait(tab_h, idxv, 1)
            compute(u + 1, 1)
            return carry

        lax.fori_loop(0, n_units // 2, body, 0)
        wait(tab_h, idxv, 0)

    return pipeline


def _sc_center_sums(ctab, cidx, B, D):
    uc = B // _U // _NW
    nd16 = D // 16
    w = 2 * D

    @functools.partial(
        pl.kernel,
        mesh=plsc.VectorSubcoreMesh(**_MESH),
        out_type=jax.ShapeDtypeStruct((B // 2, w), jnp.float32),
        scratch_types=[
            pltpu.VMEM((2, uc, _U), jnp.int32),
            pltpu.VMEM((_U // 2, w), jnp.float32),
            pltpu.VMEM((_U, w), jnp.float32),
            pltpu.VMEM((_U, w), jnp.float32),
            pltpu.VMEM((_U, w), jnp.float32),
            pltpu.VMEM((_U, w), jnp.float32),
            pltpu.SemaphoreType.DMA,
            pltpu.SemaphoreType.DMA,
        ],
        compiler_params=_PARAMS,
    )
    def sc_a(ctab_h, cidx_h, csum_h, idxc, stage, ba0, bb0, ba1, bb1,
             sem0, sem1):
        wid = lax.axis_index("s") * _NC + lax.axis_index("c")
        bufs = ((ba0, bb0, sem0), (ba1, bb1, sem1))
        pipeline = _mk_pipeline(bufs)

        base = pl.multiple_of(wid * uc, 8)
        for f in range(2):
            pltpu.sync_copy(cidx_h.at[f, pl.ds(base, uc)], idxc.at[f])

        def csum(j, p):
            ba, bb, _ = bufs[p]

            def srow(r, c2):
                row = r // 2
                colb = (r % 2) * D
                for c in range(nd16):
                    stage[row, pl.ds(colb + c * 16, 16)] = (
                        ba[r, pl.ds(c * 16, 16)]
                        + bb[r, pl.ds(D + c * 16, 16)])
                return c2

            lax.fori_loop(0, _U, srow, 0)
            rowout = pl.multiple_of((wid * uc + j) * (_U // 2), 8)
            pltpu.sync_copy(stage, csum_h.at[pl.ds(rowout, _U // 2)])

        pipeline(ctab_h, idxc, uc, csum)

    return sc_a(ctab, cidx)


def _sc_scores(xtab, csum, xidx, nidx, B, BN, D, neg):
    uc = B // _U // _NW
    un = BN // _U // _NW
    rc = uc * _U
    rn = un * _U
    w = 2 * D

    @functools.partial(
        pl.kernel,
        mesh=plsc.VectorSubcoreMesh(**_MESH),
        out_type=[
            jax.ShapeDtypeStruct((B,), jnp.float32),
            jax.ShapeDtypeStruct((BN,), jnp.float32),
        ],
        scratch_types=[
            pltpu.VMEM((2, uc, _U), jnp.int32),
            pltpu.VMEM((2, un, _U), jnp.int32),
            pltpu.VMEM((rc // 2, w), jnp.float32),
            pltpu.VMEM((_U, w), jnp.float32),
            pltpu.VMEM((_U, w), jnp.float32),
            pltpu.VMEM((_U, w), jnp.float32),
            pltpu.VMEM((_U, w), jnp.float32),
            pltpu.VMEM((rc,), jnp.float32),
            pltpu.VMEM((rn,), jnp.float32),
            pltpu.SemaphoreType.DMA,
            pltpu.SemaphoreType.DMA,
        ],
        compiler_params=_PARAMS,
    )
    def sc_b(xtab_h, csum_h, xidx_h, nidx_h, pos_h, negout_h,
             idxx, idxn, cbuf, ba0, bb0, ba1, bb1, pbuf, nbuf, sem0, sem1):
        wid = lax.axis_index("s") * _NC + lax.axis_index("c")
        iota = lax.iota(jnp.int32, 16)
        zero16 = jnp.zeros((16,), jnp.float32)
        bufs = ((ba0, bb0, sem0), (ba1, bb1, sem1))
        pipeline = _mk_pipeline(bufs)

        cbase = pl.multiple_of(wid * (rc // 2), 8)
        pltpu.sync_copy(csum_h.at[pl.ds(cbase, rc // 2)], cbuf)
        basex = pl.multiple_of(wid * uc, 8)
        basen = pl.multiple_of(wid * un, 8)
        for f in range(2):
            pltpu.sync_copy(xidx_h.at[f, pl.ds(basex, uc)], idxx.at[f])
            pltpu.sync_copy(nidx_h.at[f, pl.ds(basen, un)], idxn.at[f])

        def dots(j, p, rows_c_of, out_buf):
            ba, bb, _ = bufs[p]

            def dotg(g, c2):
                rows_u = g * 16 + iota
                bloc = rows_c_of(j * _U + rows_u)
                rows_c = lax.shift_right_logical(bloc, 1)
                colb_c = jnp.bitwise_and(bloc, 1) * D

                def dd(k4, acc):
                    for kk in range(4):
                        dcol = jnp.bitwise_and(iota + (k4 * 4 + kk), D - 1)
                        va = plsc.load_gather(cbuf, [rows_c, colb_c + dcol])
                        xa = plsc.load_gather(ba, [rows_u, dcol])
                        xb = plsc.load_gather(bb, [rows_u, dcol + D])
                        acc = acc + va * (xa + xb)
                    return acc

                acc = lax.fori_loop(0, D // 4, dd, zero16)
                out_buf[pl.ds(j * _U + g * 16, 16)] = acc
                return c2

            lax.fori_loop(0, _U // 16, dotg, 0)

        pipeline(xtab_h, idxx, uc,
                 lambda j, p: dots(j, p, lambda r: r, pbuf))
        pltpu.sync_copy(pbuf, pos_h.at[pl.ds(wid * rc, rc)])

        pipeline(xtab_h, idxn, un,
                 lambda j, p: dots(j, p, lambda r: r // neg, nbuf))
        pltpu.sync_copy(nbuf, negout_h.at[pl.ds(wid * rn, rn)])

    return sc_b(xtab, csum, xidx, nidx)


def _tc_loss(pos2d, neg2d, B):
    s = 0.0625

    def tc_fn(p_ref, n_ref, o_ref):
        ps = jnp.clip(s * p_ref[...], -10.0, 10.0)
        ns = jnp.clip(s * n_ref[...], -10.0, 10.0)
        pos_loss = jnp.sum(jnp.log(1.0 + jnp.exp(-ps)))
        neg_loss = jnp.sum(jnp.log(1.0 + jnp.exp(ns)))
        o_ref[0, 0] = (pos_loss + neg_loss) * (1.0 / B)

    out = pl.pallas_call(
        tc_fn,
        out_specs=pl.BlockSpec(memory_space=pltpu.SMEM),
        out_shape=jax.ShapeDtypeStruct((1, 1), jnp.float32),
    )(pos2d, neg2d)
    return out[0, 0]


def kernel(centers, contexts, neg_contexts, center_emb, context_emb):
    F, V, D = center_emb.shape
    B = centers.shape[0]
    BN = neg_contexts.shape[0]
    neg = BN // B

    ctab = jnp.transpose(center_emb, (1, 0, 2)).reshape(V, F * D)
    xtab = jnp.transpose(context_emb, (1, 0, 2)).reshape(V, F * D)
    cidx = centers.T.reshape(F, B // _U, _U)
    xidx = contexts.T.reshape(F, B // _U, _U)
    nidx = neg_contexts.T.reshape(F, BN // _U, _U)

    csum = _sc_center_sums(ctab, cidx, B, D)
    pos, negraw = _sc_scores(xtab, csum, xidx, nidx, B, BN, D, neg)
    return _tc_loss(pos.reshape(B // 128, 128), negraw.reshape(BN // 128, 128), B)


# speedup vs baseline: 3.0567x; 1.0364x over previous
import functools

import jax
import jax.numpy as jnp
from jax import lax
from jax.experimental import pallas as pl
from jax.experimental.pallas import tpu as pltpu
from jax.experimental.pallas import tpu_sc as plsc

_NC = 2
_NS = 16
_NW = _NC * _NS
_U = 64

_MESH = dict(core_axis_name="c", subcore_axis_name="s")
_PARAMS = pltpu.CompilerParams(use_tc_tiling_on_sc=True,
                               needs_layout_passes=False)


def _mk_pipeline(bufs):
    def fire(tab_h, idxv, j, p):
        ba, bb, sem = bufs[p]
        pltpu.async_copy(tab_h.at[idxv.at[0, j]], ba, sem)
        pltpu.async_copy(tab_h.at[idxv.at[1, j]], bb, sem)

    def wait(tab_h, idxv, p):
        ba, bb, sem = bufs[p]
        pltpu.make_async_copy(tab_h.at[idxv.at[0, 0]], ba, sem).wait()
        pltpu.make_async_copy(tab_h.at[idxv.at[1, 0]], bb, sem).wait()

    def pipeline(tab_h, idxv, n_units, compute):
        fire(tab_h, idxv, 0, 0)

        def body(j2, carry):
            u = 2 * j2
            fire(tab_h, idxv, u + 1, 1)
            wait(tab_h, idxv, 0)
            compute(u, 0)
            fire(tab_h, idxv, jnp.minimum(u + 2, n_units - 1), 0)
            wait(tab_h, idxv, 1)
            compute(u + 1, 1)
            return carry

        lax.fori_loop(0, n_units // 2, body, 0)
        wait(tab_h, idxv, 0)

    return pipeline


def _sc_center_sums(ctab, cidx, B, D):
    uc = B // _U // _NW
    nd16 = D // 16
    w = 2 * D

    @functools.partial(
        pl.kernel,
        mesh=plsc.VectorSubcoreMesh(**_MESH),
        out_type=jax.ShapeDtypeStruct((B // 2, w), jnp.float32),
        scratch_types=[
            pltpu.VMEM((2, uc, _U), jnp.int32),
            pltpu.VMEM((_U // 2, w), jnp.float32),
            pltpu.VMEM((_U, w), jnp.float32),
            pltpu.VMEM((_U, w), jnp.float32),
            pltpu.VMEM((_U, w), jnp.float32),
            pltpu.VMEM((_U, w), jnp.float32),
            pltpu.SemaphoreType.DMA,
            pltpu.SemaphoreType.DMA,
        ],
        compiler_params=_PARAMS,
    )
    def sc_a(ctab_h, cidx_h, csum_h, idxc, stage, ba0, bb0, ba1, bb1,
             sem0, sem1):
        wid = lax.axis_index("s") * _NC + lax.axis_index("c")
        bufs = ((ba0, bb0, sem0), (ba1, bb1, sem1))
        pipeline = _mk_pipeline(bufs)

        base = pl.multiple_of(wid * uc, 8)
        for f in range(2):
            pltpu.sync_copy(cidx_h.at[f, pl.ds(base, uc)], idxc.at[f])

        def csum(j, p):
            ba, bb, _ = bufs[p]

            def srow(r, c2):
                row = r // 2
                colb = (r % 2) * D
                for c in range(nd16):
                    stage[row, pl.ds(colb + c * 16, 16)] = (
                        ba[r, pl.ds(c * 16, 16)]
                        + bb[r, pl.ds(D + c * 16, 16)])
                return c2

            lax.fori_loop(0, _U, srow, 0)
            rowout = pl.multiple_of((wid * uc + j) * (_U // 2), 8)
            pltpu.sync_copy(stage, csum_h.at[pl.ds(rowout, _U // 2)])

        pipeline(ctab_h, idxc, uc, csum)

    return sc_a(ctab, cidx)


def _sc_scores(xtab, csum, xidx, nidx, B, BN, D, neg):
    ub = 2 * _U
    uc = B // ub // _NW
    un = BN // ub // _NW
    rc = uc * ub
    rn = un * ub
    w = 2 * D

    @functools.partial(
        pl.kernel,
        mesh=plsc.VectorSubcoreMesh(**_MESH),
        out_type=[
            jax.ShapeDtypeStruct((B,), jnp.float32),
            jax.ShapeDtypeStruct((BN,), jnp.float32),
        ],
        scratch_types=[
            pltpu.VMEM((2, uc + 4, ub), jnp.int32),
            pltpu.VMEM((2, un + 4, ub), jnp.int32),
            pltpu.VMEM((rc // 2, w), jnp.float32),
            pltpu.VMEM((ub, w), jnp.float32),
            pltpu.VMEM((ub, w), jnp.float32),
            pltpu.VMEM((ub, w), jnp.float32),
            pltpu.VMEM((ub, w), jnp.float32),
            pltpu.VMEM((rc,), jnp.float32),
            pltpu.VMEM((rn,), jnp.float32),
            pltpu.SemaphoreType.DMA,
            pltpu.SemaphoreType.DMA,
        ],
        compiler_params=_PARAMS,
    )
    def sc_b(xtab_h, csum_h, xidx_h, nidx_h, pos_h, negout_h,
             idxx, idxn, cbuf, ba0, bb0, ba1, bb1, pbuf, nbuf, sem0, sem1):
        wid = lax.axis_index("s") * _NC + lax.axis_index("c")
        iota = lax.iota(jnp.int32, 16)
        zero16 = jnp.zeros((16,), jnp.float32)
        bufs = ((ba0, bb0, sem0), (ba1, bb1, sem1))
        pipeline = _mk_pipeline(bufs)

        cbase = pl.multiple_of(wid * (rc // 2), 8)
        pltpu.sync_copy(csum_h.at[pl.ds(cbase, rc // 2)], cbuf)
        offx = lax.rem(wid * uc, 8)
        offn = lax.rem(wid * un, 8)
        basex = pl.multiple_of(wid * uc - offx, 8)
        basen = pl.multiple_of(wid * un - offn, 8)
        for f in range(2):
            pltpu.sync_copy(xidx_h.at[f, pl.ds(basex, uc + 4)], idxx.at[f])
            pltpu.sync_copy(nidx_h.at[f, pl.ds(basen, un + 4)], idxn.at[f])

        def dots(j, p, rows_c_of, out_buf):
            ba, bb, _ = bufs[p]

            def dotg(g, c2):
                rows_u = g * 16 + iota
                bloc = rows_c_of(j * ub + rows_u)
                rows_c = lax.shift_right_logical(bloc, 1)
                colb_c = jnp.bitwise_and(bloc, 1) * D

                def dd(k4, acc):
                    for kk in range(4):
                        dcol = jnp.bitwise_and(iota + (k4 * 4 + kk), D - 1)
                        va = plsc.load_gather(cbuf, [rows_c, colb_c + dcol])
                        xa = plsc.load_gather(ba, [rows_u, dcol])
                        xb = plsc.load_gather(bb, [rows_u, dcol + D])
                        acc = acc + va * (xa + xb)
                    return acc

                acc = lax.fori_loop(0, D // 4, dd, zero16)
                out_buf[pl.ds(j * ub + g * 16, 16)] = acc
                return c2

            lax.fori_loop(0, ub // 16, dotg, 0)

        def mkidx(idxv, off):
            class _Sl:
                def __init__(self, ref):
                    self.ref = ref
                @property
                def at(self):
                    return self
                def __getitem__(self, key):
                    f, j = key
                    return idxv.at[f, off + j]
            return _Sl(idxv)

        pipeline(xtab_h, mkidx(idxx, offx), uc,
                 lambda j, p: dots(j, p, lambda r: r, pbuf))
        pltpu.sync_copy(pbuf, pos_h.at[pl.ds(wid * rc, rc)])

        pipeline(xtab_h, mkidx(idxn, offn), un,
                 lambda j, p: dots(j, p, lambda r: r // neg, nbuf))
        pltpu.sync_copy(nbuf, negout_h.at[pl.ds(wid * rn, rn)])

    return sc_b(xtab, csum, xidx, nidx)


def _tc_loss(pos2d, neg2d, B):
    s = 0.0625

    def tc_fn(p_ref, n_ref, o_ref):
        ps = jnp.clip(s * p_ref[...], -10.0, 10.0)
        ns = jnp.clip(s * n_ref[...], -10.0, 10.0)
        pos_loss = jnp.sum(jnp.log(1.0 + jnp.exp(-ps)))
        neg_loss = jnp.sum(jnp.log(1.0 + jnp.exp(ns)))
        o_ref[0, 0] = (pos_loss + neg_loss) * (1.0 / B)

    out = pl.pallas_call(
        tc_fn,
        out_specs=pl.BlockSpec(memory_space=pltpu.SMEM),
        out_shape=jax.ShapeDtypeStruct((1, 1), jnp.float32),
    )(pos2d, neg2d)
    return out[0, 0]


def kernel(centers, contexts, neg_contexts, center_emb, context_emb):
    F, V, D = center_emb.shape
    B = centers.shape[0]
    BN = neg_contexts.shape[0]
    neg = BN // B

    ctab = jnp.transpose(center_emb, (1, 0, 2)).reshape(V, F * D)
    xtab = jnp.transpose(context_emb, (1, 0, 2)).reshape(V, F * D)
    cidx = centers.T.reshape(F, B // _U, _U)
    xidx = contexts.T.reshape(F, B // (2 * _U), 2 * _U)
    nidx = neg_contexts.T.reshape(F, BN // (2 * _U), 2 * _U)

    csum = _sc_center_sums(ctab, cidx, B, D)
    pos, negraw = _sc_scores(xtab, csum, xidx, nidx, B, BN, D, neg)
    return _tc_loss(pos.reshape(B // 128, 128), negraw.reshape(BN // 128, 128), B)
